# baseline (speedup 1.0000x reference)
import functools

import jax
import jax.numpy as jnp
from jax import lax
from jax.experimental import pallas as pl
from jax.experimental.pallas import tpu as pltpu
from jax.experimental.pallas import tpu_sc as plsc

_NC = 2
_NS = 16
_NW = _NC * _NS

_B = 16384
_D = 64
_H = 128
_BPW = _B // _NW
_CHUNK = 128
_NCHUNK = _BPW // _CHUNK


def _sc_gather(users2d, courses2d, user_table, course_table):
  mesh = plsc.VectorSubcoreMesh(core_axis_name="c", subcore_axis_name="s")

  @functools.partial(
      pl.kernel,
      out_type=(
          jax.ShapeDtypeStruct((_B, _D), jnp.float32),
          jax.ShapeDtypeStruct((_B, _D), jnp.float32),
      ),
      mesh=mesh,
      compiler_params=pltpu.CompilerParams(use_tc_tiling_on_sc=False),
      scratch_types=[
          pltpu.VMEM((_NCHUNK, _CHUNK), jnp.int32),
          pltpu.VMEM((_NCHUNK, _CHUNK), jnp.int32),
          pltpu.VMEM((_BPW, _D), jnp.float32),
          pltpu.VMEM((_BPW, _D), jnp.float32),
          pltpu.SemaphoreType.DMA,
      ],
  )
  def k(uidx_hbm, cidx_hbm, ut_hbm, ct_hbm, u_out, c_out,
        uidx_v, cidx_v, urows_v, crows_v, sem):
    wid = lax.axis_index("s") * _NC + lax.axis_index("c")
    base = wid * _BPW
    pltpu.sync_copy(uidx_hbm.at[pl.ds(wid * _NCHUNK, _NCHUNK)], uidx_v)
    pltpu.sync_copy(cidx_hbm.at[pl.ds(wid * _NCHUNK, _NCHUNK)], cidx_v)
    copies = []
    for j in range(_NCHUNK):
      copies.append(pltpu.async_copy(
          ut_hbm.at[uidx_v.at[j]], urows_v.at[pl.ds(j * _CHUNK, _CHUNK)], sem))
      copies.append(pltpu.async_copy(
          ct_hbm.at[cidx_v.at[j]], crows_v.at[pl.ds(j * _CHUNK, _CHUNK)], sem))
    for cp in copies:
      cp.wait()
    pltpu.sync_copy(urows_v, u_out.at[pl.ds(base, _BPW)])
    pltpu.sync_copy(crows_v, c_out.at[pl.ds(base, _BPW)])

  return k(users2d, courses2d, user_table, course_table)


def _mlp_body(u_ref, c_ref, w1u_ref, w1c_ref, b1_ref, w2_ref, b2_ref, o_ref):
  h = lax.dot_general(u_ref[...], w1u_ref[...], (((1,), (1,)), ((), ())),
                      preferred_element_type=jnp.float32)
  h = h + lax.dot_general(c_ref[...], w1c_ref[...], (((1,), (1,)), ((), ())),
                          preferred_element_type=jnp.float32)
  h = jnp.maximum(h + b1_ref[...], 0.0)
  o_ref[...] = jnp.sum(h * w2_ref[...], axis=1, keepdims=True) + b2_ref[...]


def _tc_mlp(u, c, w1u, w1c, b1, w2, b2):
  rows = 2048
  grid = _B // rows
  return pl.pallas_call(
      _mlp_body,
      grid=(grid,),
      in_specs=[
          pl.BlockSpec((rows, _D), lambda i: (i, 0)),
          pl.BlockSpec((rows, _D), lambda i: (i, 0)),
          pl.BlockSpec((_H, _D), lambda i: (0, 0)),
          pl.BlockSpec((_H, _D), lambda i: (0, 0)),
          pl.BlockSpec((1, _H), lambda i: (0, 0)),
          pl.BlockSpec((1, _H), lambda i: (0, 0)),
          pl.BlockSpec((1, 1), lambda i: (0, 0)),
      ],
      out_specs=pl.BlockSpec((rows, 1), lambda i: (i, 0)),
      out_shape=jax.ShapeDtypeStruct((_B, 1), jnp.float32),
  )(u, c, w1u, w1c, b1, w2, b2)


def kernel(users, courses, user_table, course_table, W1, b1, W2, b2):
  users2d = users.astype(jnp.int32).reshape(_NW * _NCHUNK, _CHUNK)
  courses2d = courses.astype(jnp.int32).reshape(_NW * _NCHUNK, _CHUNK)
  u, c = _sc_gather(users2d, courses2d, user_table, course_table)
  return _tc_mlp(u, c, W1[:, :_D], W1[:, _D:], b1.reshape(1, _H),
                 W2, b2.reshape(1, 1))
